# baseline (speedup 1.0000x reference)
import jax
import jax.numpy as jnp
from jax import lax
from jax.experimental import pallas as pl
from jax.experimental.pallas import tpu as pltpu
from jax.experimental.pallas import tpu_sc as plsc

N_EMB = 16384
D = 512
DP = D // 2
N_TRIPLETS = 200000
MARGIN = 1.0

NW = 32
T_LO = 6248
T_HI = 6256
RAW_W = T_HI * 3
CHUNK = 32
N_CHUNKS = 196


def _tl_body(emb_hbm, trip_hbm, out_hbm,
             raw_v,
             i0a_buf, i1a_buf, i2a_buf, i0b_buf, i1b_buf, i2b_buf,
             a0_buf, p0_buf, n0_buf, a1_buf, p1_buf, n1_buf,
             loss_v, sem0, sem1):
    cid = lax.axis_index("c")
    sid = lax.axis_index("s")
    wid = sid * 2 + cid
    n_t = jnp.where(wid < 24, T_LO, T_HI)
    base = wid * T_LO + jnp.maximum(wid - 24, 0) * 8

    pltpu.sync_copy(trip_hbm.at[pl.ds(base, T_HI)], raw_v)

    lanes = lax.iota(jnp.int32, 16)
    lanes_hi = lanes + 16
    zero16 = jnp.zeros((16,), jnp.float32)
    max_slot = T_HI - 1

    def issue(g, ia, ip, iq, a_b, p_b, n_b, sem):
        off = g * CHUNK
        slot_lo = jnp.minimum(off + lanes, max_slot)
        slot_hi = jnp.minimum(off + lanes_hi, max_slot)
        for col_id, ibuf in ((0, ia), (1, ip), (2, iq)):
            col = jnp.full((16,), col_id, jnp.int32)
            ibuf[pl.ds(0, 16)] = plsc.load_gather(raw_v, [slot_lo, col])
            ibuf[pl.ds(16, 16)] = plsc.load_gather(raw_v, [slot_hi, col])
        pltpu.async_copy(emb_hbm.at[ia], a_b, sem)
        pltpu.async_copy(emb_hbm.at[ip], p_b, sem)
        pltpu.async_copy(emb_hbm.at[iq], n_b, sem)

    def drain(a_b, p_b, n_b, ia, sem):
        pltpu.make_async_copy(emb_hbm.at[ia], a_b, sem).wait()
        pltpu.make_async_copy(emb_hbm.at[ia], p_b, sem).wait()
        pltpu.make_async_copy(emb_hbm.at[ia], n_b, sem).wait()

    def compute(g, a_b, p_b, n_b, loss_acc):
        def pair_terms(wa, wp, wn):
            ba = plsc.bitcast(wa, jnp.bfloat16)
            bp = plsc.bitcast(wp, jnp.bfloat16)
            bn = plsc.bitcast(wn, jnp.bfloat16)
            d = bp - bn
            f = (bp + bn) - (ba + ba)
            prod = d * f
            return plsc.unpack(prod, format=plsc.PackFormat.INTERLEAVED,
                               preferred_element_type=jnp.float32)

        def d_body(c, carry):
            acc0, acc1, acc2, acc3 = carry
            col = (lanes + c) & (DP - 1)
            a0 = plsc.load_gather(a_b, [lanes, col])
            p0 = plsc.load_gather(p_b, [lanes, col])
            n0 = plsc.load_gather(n_b, [lanes, col])
            a1 = plsc.load_gather(a_b, [lanes_hi, col])
            p1 = plsc.load_gather(p_b, [lanes_hi, col])
            n1 = plsc.load_gather(n_b, [lanes_hi, col])
            lo0, hi0 = pair_terms(a0, p0, n0)
            lo1, hi1 = pair_terms(a1, p1, n1)
            return (acc0 + lo0, acc1 + hi0, acc2 + lo1, acc3 + hi1)

        acc0, acc1, acc2, acc3 = lax.fori_loop(
            0, DP, d_body, (zero16, zero16, zero16, zero16), unroll=4)
        slot = g * CHUNK
        l0 = jnp.maximum((acc0 + acc1) + MARGIN, 0.0)
        l1 = jnp.maximum((acc2 + acc3) + MARGIN, 0.0)
        l0 = jnp.where(slot + lanes < n_t, l0, 0.0)
        l1 = jnp.where(slot + lanes_hi < n_t, l1, 0.0)
        return loss_acc + l0 + l1

    issue(0, i0a_buf, i1a_buf, i2a_buf, a0_buf, p0_buf, n0_buf, sem0)

    def pair_body(h, loss_acc):
        g0 = 2 * h
        issue(g0 + 1, i0b_buf, i1b_buf, i2b_buf, a1_buf, p1_buf, n1_buf, sem1)
        drain(a0_buf, p0_buf, n0_buf, i0a_buf, sem0)
        loss_acc = compute(g0, a0_buf, p0_buf, n0_buf, loss_acc)

        @pl.when(g0 + 2 < N_CHUNKS)
        def _():
            issue(g0 + 2, i0a_buf, i1a_buf, i2a_buf,
                  a0_buf, p0_buf, n0_buf, sem0)

        drain(a1_buf, p1_buf, n1_buf, i0b_buf, sem1)
        return compute(g0 + 1, a1_buf, p1_buf, n1_buf, loss_acc)

    loss_acc = lax.fori_loop(0, N_CHUNKS // 2, pair_body, zero16)
    loss_v[...] = loss_acc
    pltpu.sync_copy(loss_v, out_hbm.at[wid])


@jax.jit
def _tl_call(emb_packed, trip_flat):
    mesh = plsc.VectorSubcoreMesh(core_axis_name="c", subcore_axis_name="s")
    row = pltpu.VMEM((CHUNK, DP), jnp.int32)
    idxb = pltpu.VMEM((CHUNK,), jnp.int32)
    fn = pl.kernel(
        _tl_body,
        out_type=jax.ShapeDtypeStruct((NW, 16), jnp.float32),
        mesh=mesh,
        scratch_types=[
            pltpu.VMEM((T_HI, 3), jnp.int32),
            idxb, idxb, idxb, idxb, idxb, idxb,
            row, row, row, row, row, row,
            pltpu.VMEM((16,), jnp.float32),
            pltpu.SemaphoreType.DMA,
            pltpu.SemaphoreType.DMA,
        ],
        compiler_params=pltpu.CompilerParams(
            use_tc_tiling_on_sc=False, needs_layout_passes=False),
    )
    return fn(emb_packed, trip_flat)


def _pack_table(emb):
    u = jax.lax.bitcast_convert_type(emb, jnp.uint32)
    rne = lambda w: (w + jnp.uint32(0x7FFF) + ((w >> 16) & 1)) >> 16
    packed = rne(u[:, 0::2]) | (rne(u[:, 1::2]) << 16)
    return jax.lax.bitcast_convert_type(packed, jnp.int32)


def kernel(embeddings, target, triplets):
    del target
    partials = _tl_call(_pack_table(embeddings), triplets)
    return (jnp.sum(partials) / N_TRIPLETS, triplets.shape[0])


# speedup vs baseline: 3.1248x; 3.1248x over previous
import jax
import jax.numpy as jnp
from jax import lax
from jax.experimental import pallas as pl
from jax.experimental.pallas import tpu as pltpu
from jax.experimental.pallas import tpu_sc as plsc

N_EMB = 16384
D = 512
DP = D // 2
N_TRIPLETS = 200000
MARGIN = 1.0

NW = 32
T_LO = 6248
T_HI = 6256
RAW_W = T_HI * 3
CHUNK = 32
N_CHUNKS = 196


def _tl_body(emb_hbm, trip_hbm, out_hbm,
             raw_v,
             i0a_buf, i1a_buf, i2a_buf, i0b_buf, i1b_buf, i2b_buf,
             a0_buf, p0_buf, n0_buf, a1_buf, p1_buf, n1_buf,
             loss_v, sem0, sem1):
    cid = lax.axis_index("c")
    sid = lax.axis_index("s")
    wid = sid * 2 + cid
    n_t = jnp.where(wid < 24, T_LO, T_HI)
    base = wid * T_LO + jnp.maximum(wid - 24, 0) * 8

    pltpu.sync_copy(trip_hbm.at[pl.ds(base, T_HI)], raw_v)

    lanes = lax.iota(jnp.int32, 16)
    lanes_hi = lanes + 16
    zero16 = jnp.zeros((16,), jnp.float32)
    max_slot = T_HI - 1

    def issue(g, ia, ip, iq, a_b, p_b, n_b, sem):
        off = g * CHUNK
        slot_lo = jnp.minimum(off + lanes, max_slot)
        slot_hi = jnp.minimum(off + lanes_hi, max_slot)
        for col_id, ibuf in ((0, ia), (1, ip), (2, iq)):
            col = jnp.full((16,), col_id, jnp.int32)
            ibuf[pl.ds(0, 16)] = plsc.load_gather(raw_v, [slot_lo, col])
            ibuf[pl.ds(16, 16)] = plsc.load_gather(raw_v, [slot_hi, col])
        pltpu.async_copy(emb_hbm.at[ia], a_b, sem)
        pltpu.async_copy(emb_hbm.at[ip], p_b, sem)
        pltpu.async_copy(emb_hbm.at[iq], n_b, sem)

    def drain(a_b, p_b, n_b, ia, sem):
        pltpu.make_async_copy(emb_hbm.at[ia], a_b, sem).wait()
        pltpu.make_async_copy(emb_hbm.at[ia], p_b, sem).wait()
        pltpu.make_async_copy(emb_hbm.at[ia], n_b, sem).wait()

    def compute(g, a_b, p_b, n_b, loss_acc):
        def pair_terms(wa, wp, wn):
            ba = plsc.bitcast(wa, jnp.bfloat16)
            bp = plsc.bitcast(wp, jnp.bfloat16)
            bn = plsc.bitcast(wn, jnp.bfloat16)
            d = bp - bn
            f = (bp + bn) - (ba + ba)
            prod = d * f
            return plsc.unpack(prod, format=plsc.PackFormat.INTERLEAVED,
                               preferred_element_type=jnp.float32)

        def d_body(c, carry):
            acc0, acc1, acc2, acc3 = carry
            col = (lanes + c) & (DP - 1)
            a0 = plsc.load_gather(a_b, [lanes, col])
            p0 = plsc.load_gather(p_b, [lanes, col])
            n0 = plsc.load_gather(n_b, [lanes, col])
            a1 = plsc.load_gather(a_b, [lanes_hi, col])
            p1 = plsc.load_gather(p_b, [lanes_hi, col])
            n1 = plsc.load_gather(n_b, [lanes_hi, col])
            lo0, hi0 = pair_terms(a0, p0, n0)
            lo1, hi1 = pair_terms(a1, p1, n1)
            return (acc0 + lo0, acc1 + hi0, acc2 + lo1, acc3 + hi1)

        acc0, acc1, acc2, acc3 = lax.fori_loop(
            0, DP, d_body, (zero16, zero16, zero16, zero16), unroll=4)
        slot = g * CHUNK
        l0 = jnp.maximum((acc0 + acc1) + MARGIN, 0.0)
        l1 = jnp.maximum((acc2 + acc3) + MARGIN, 0.0)
        l0 = jnp.where(slot + lanes < n_t, l0, 0.0)
        l1 = jnp.where(slot + lanes_hi < n_t, l1, 0.0)
        return loss_acc + l0 + l1

    issue(0, i0a_buf, i1a_buf, i2a_buf, a0_buf, p0_buf, n0_buf, sem0)

    def pair_body(h, loss_acc):
        g0 = 2 * h
        issue(g0 + 1, i0b_buf, i1b_buf, i2b_buf, a1_buf, p1_buf, n1_buf, sem1)
        drain(a0_buf, p0_buf, n0_buf, i0a_buf, sem0)
        loss_acc = compute(g0, a0_buf, p0_buf, n0_buf, loss_acc)

        @pl.when(g0 + 2 < N_CHUNKS)
        def _():
            issue(g0 + 2, i0a_buf, i1a_buf, i2a_buf,
                  a0_buf, p0_buf, n0_buf, sem0)

        drain(a1_buf, p1_buf, n1_buf, i0b_buf, sem1)
        return compute(g0 + 1, a1_buf, p1_buf, n1_buf, loss_acc)

    loss_acc = lax.fori_loop(0, N_CHUNKS // 2, pair_body, zero16)
    loss_v[...] = loss_acc
    pltpu.sync_copy(loss_v, out_hbm.at[wid])


@jax.jit
def _tl_call(emb_packed, trip_flat):
    mesh = plsc.VectorSubcoreMesh(core_axis_name="c", subcore_axis_name="s")
    row = pltpu.VMEM((CHUNK, DP), jnp.int32)
    idxb = pltpu.VMEM((CHUNK,), jnp.int32)
    fn = pl.kernel(
        _tl_body,
        out_type=jax.ShapeDtypeStruct((NW, 16), jnp.float32),
        mesh=mesh,
        scratch_types=[
            pltpu.VMEM((T_HI, 3), jnp.int32),
            idxb, idxb, idxb, idxb, idxb, idxb,
            row, row, row, row, row, row,
            pltpu.VMEM((16,), jnp.float32),
            pltpu.SemaphoreType.DMA,
            pltpu.SemaphoreType.DMA,
        ],
        compiler_params=pltpu.CompilerParams(
            use_tc_tiling_on_sc=False, needs_layout_passes=False),
    )
    return fn(emb_packed, trip_flat)


PBLK = 512


def _pack_body(x_ref, o_ref):
    u = jax.lax.bitcast_convert_type(x_ref[...], jnp.uint32)
    rne = lambda w: (w + jnp.uint32(0x7FFF) + ((w >> 16) & 1)) >> 16
    packed = rne(u[:, :DP]) | (rne(u[:, DP:]) << 16)
    o_ref[...] = jax.lax.bitcast_convert_type(packed, jnp.int32)


def _pack_table(emb):
    return pl.pallas_call(
        _pack_body,
        grid=(N_EMB // PBLK,),
        in_specs=[pl.BlockSpec((PBLK, D), lambda i: (i, 0))],
        out_specs=pl.BlockSpec((PBLK, DP), lambda i: (i, 0)),
        out_shape=jax.ShapeDtypeStruct((N_EMB, DP), jnp.int32),
    )(emb)


def kernel(embeddings, target, triplets):
    del target
    partials = _tl_call(_pack_table(embeddings), triplets)
    return (jnp.sum(partials) / N_TRIPLETS, triplets.shape[0])


# speedup vs baseline: 3.4163x; 1.0933x over previous
import jax
import jax.numpy as jnp
from jax import lax
from jax.experimental import pallas as pl
from jax.experimental.pallas import tpu as pltpu
from jax.experimental.pallas import tpu_sc as plsc

N_EMB = 16384
D = 512
DP = D // 2
N_TRIPLETS = 200000
MARGIN = 1.0

NW = 32
T_LO = 6248
T_HI = 6256
RAW_W = T_HI * 3
CHUNK = 32
N_CHUNKS = 198


def _tl_body(emb_hbm, trip_hbm, out_hbm,
             raw_v,
             ia0, ip0, iq0, ia1, ip1, iq1, ia2, ip2, iq2,
             a0, p0, n0, a1, p1, n1, a2, p2, n2,
             loss_v, sem0, sem1, sem2):
    idx_slots = [(ia0, ip0, iq0), (ia1, ip1, iq1), (ia2, ip2, iq2)]
    row_slots = [(a0, p0, n0), (a1, p1, n1), (a2, p2, n2)]
    sems = [sem0, sem1, sem2]
    cid = lax.axis_index("c")
    sid = lax.axis_index("s")
    wid = sid * 2 + cid
    n_t = jnp.where(wid < 24, T_LO, T_HI)
    base = wid * T_LO + jnp.maximum(wid - 24, 0) * 8

    pltpu.sync_copy(trip_hbm.at[pl.ds(base, T_HI)], raw_v)

    lanes = lax.iota(jnp.int32, 16)
    lanes_hi = lanes + 16
    zero16 = jnp.zeros((16,), jnp.float32)
    max_slot = T_HI - 1

    def issue(g, ia, ip, iq, a_b, p_b, n_b, sem):
        off = g * CHUNK
        slot_lo = jnp.minimum(off + lanes, max_slot)
        slot_hi = jnp.minimum(off + lanes_hi, max_slot)
        for col_id, ibuf in ((0, ia), (1, ip), (2, iq)):
            col = jnp.full((16,), col_id, jnp.int32)
            ibuf[pl.ds(0, 16)] = plsc.load_gather(raw_v, [slot_lo, col])
            ibuf[pl.ds(16, 16)] = plsc.load_gather(raw_v, [slot_hi, col])
        pltpu.async_copy(emb_hbm.at[ia], a_b, sem)
        pltpu.async_copy(emb_hbm.at[ip], p_b, sem)
        pltpu.async_copy(emb_hbm.at[iq], n_b, sem)

    def drain(a_b, p_b, n_b, ia, sem):
        pltpu.make_async_copy(emb_hbm.at[ia], a_b, sem).wait()
        pltpu.make_async_copy(emb_hbm.at[ia], p_b, sem).wait()
        pltpu.make_async_copy(emb_hbm.at[ia], n_b, sem).wait()

    def compute(g, a_b, p_b, n_b, loss_acc):
        def pair_terms(wa, wp, wn):
            ba = plsc.bitcast(wa, jnp.bfloat16)
            bp = plsc.bitcast(wp, jnp.bfloat16)
            bn = plsc.bitcast(wn, jnp.bfloat16)
            d = bp - bn
            f = (bp + bn) - (ba + ba)
            prod = d * f
            return plsc.unpack(prod, format=plsc.PackFormat.INTERLEAVED,
                               preferred_element_type=jnp.float32)

        def d_body(c, carry):
            acc0, acc1, acc2, acc3 = carry
            col = (lanes + c) & (DP - 1)
            a0 = plsc.load_gather(a_b, [lanes, col])
            p0 = plsc.load_gather(p_b, [lanes, col])
            n0 = plsc.load_gather(n_b, [lanes, col])
            a1 = plsc.load_gather(a_b, [lanes_hi, col])
            p1 = plsc.load_gather(p_b, [lanes_hi, col])
            n1 = plsc.load_gather(n_b, [lanes_hi, col])
            lo0, hi0 = pair_terms(a0, p0, n0)
            lo1, hi1 = pair_terms(a1, p1, n1)
            return (acc0 + lo0, acc1 + hi0, acc2 + lo1, acc3 + hi1)

        acc0, acc1, acc2, acc3 = lax.fori_loop(
            0, DP, d_body, (zero16, zero16, zero16, zero16), unroll=4)
        slot = g * CHUNK
        l0 = jnp.maximum((acc0 + acc1) + MARGIN, 0.0)
        l1 = jnp.maximum((acc2 + acc3) + MARGIN, 0.0)
        l0 = jnp.where(slot + lanes < n_t, l0, 0.0)
        l1 = jnp.where(slot + lanes_hi < n_t, l1, 0.0)
        return loss_acc + l0 + l1

    for s in range(3):
        issue(s, *idx_slots[s], *row_slots[s], sems[s])

    def tri_body(q, loss_acc):
        g0 = 3 * q
        for s in range(3):
            g = g0 + s
            drain(*row_slots[s], idx_slots[s][0], sems[s])
            loss_acc = compute(g, *row_slots[s], loss_acc)

            @pl.when(g + 3 < N_CHUNKS)
            def _(g=g, s=s):
                issue(g + 3, *idx_slots[s], *row_slots[s], sems[s])

        return loss_acc

    loss_acc = lax.fori_loop(0, N_CHUNKS // 3, tri_body, zero16)
    loss_v[...] = loss_acc
    pltpu.sync_copy(loss_v, out_hbm.at[wid])


@jax.jit
def _tl_call(emb_packed, trip_flat):
    mesh = plsc.VectorSubcoreMesh(core_axis_name="c", subcore_axis_name="s")
    row = pltpu.VMEM((CHUNK, DP), jnp.int32)
    idxb = pltpu.VMEM((CHUNK,), jnp.int32)
    fn = pl.kernel(
        _tl_body,
        out_type=jax.ShapeDtypeStruct((NW, 16), jnp.float32),
        mesh=mesh,
        scratch_types=(
            [pltpu.VMEM((T_HI, 3), jnp.int32)]
            + [idxb] * 9
            + [row] * 9
            + [pltpu.VMEM((16,), jnp.float32)]
            + [pltpu.SemaphoreType.DMA] * 3
        ),
        compiler_params=pltpu.CompilerParams(
            use_tc_tiling_on_sc=False, needs_layout_passes=False),
    )
    return fn(emb_packed, trip_flat)


PBLK = 512


def _pack_body(x_ref, o_ref):
    u = jax.lax.bitcast_convert_type(x_ref[...], jnp.uint32)
    rne = lambda w: (w + jnp.uint32(0x7FFF) + ((w >> 16) & 1)) >> 16
    packed = rne(u[:, :DP]) | (rne(u[:, DP:]) << 16)
    o_ref[...] = jax.lax.bitcast_convert_type(packed, jnp.int32)


def _pack_table(emb):
    return pl.pallas_call(
        _pack_body,
        grid=(N_EMB // PBLK,),
        in_specs=[pl.BlockSpec((PBLK, D), lambda i: (i, 0))],
        out_specs=pl.BlockSpec((PBLK, DP), lambda i: (i, 0)),
        out_shape=jax.ShapeDtypeStruct((N_EMB, DP), jnp.int32),
    )(emb)


def kernel(embeddings, target, triplets):
    del target
    partials = _tl_call(_pack_table(embeddings), triplets)
    return (jnp.sum(partials) / N_TRIPLETS, triplets.shape[0])
